# Initial kernel scaffold; baseline (speedup 1.0000x reference)
import jax
import jax.numpy as jnp
from jax.experimental import pallas as pl


def kernel(positions, time, node_features, params):
    raise NotImplementedError("write your pallas kernel here")


# speedup vs baseline: 1.7906x; 1.7906x over previous
import functools

import jax
import jax.numpy as jnp
from jax import lax
from jax.experimental import pallas as pl
from jax.experimental.pallas import tpu as pltpu

B = 64
N = 64
DIM = 3
N_FEATURES = 16
HID = 64
TEMB = 32
N_BLOCKS = 2

_LOG1E4 = 9.210340371976184


def _silu(x):
    return x * jax.nn.sigmoid(x)


def _dot(a, b):
    return jnp.dot(a, b, preferred_element_type=jnp.float32)


def _dotx(a, b):
    return jnp.dot(a, b, preferred_element_type=jnp.float32,
                   precision=jax.lax.Precision.HIGHEST)


def _egnn_kernel(x_ref, t_ref, nf_ref, emb_ref, w_ref, b_ref, wte_ref,
                 wsq_ref, vx3_ref, cx3_ref, out_ref):
    x0 = x_ref[0]
    t = t_ref[0, 0, 0]
    nf = nf_ref[0]

    feat_iota = lax.broadcasted_iota(jnp.int32, (N, N_FEATURES), 1)
    onehot = (nf == feat_iota).astype(jnp.float32)
    h = _dotx(onehot, emb_ref[:, :])

    lane = lax.broadcasted_iota(jnp.int32, (1, TEMB), 1)
    half = TEMB // 2
    k = jnp.where(lane < half, lane, lane - half).astype(jnp.float32)
    freq = jnp.exp(k * (-_LOG1E4 / (half - 1)))
    arg = (t * 1000.0) * freq
    te = jnp.where(lane < half, jnp.sin(arg), jnp.cos(arg))

    row_i = lax.broadcasted_iota(jnp.int32, (N, N), 0)
    col_j = lax.broadcasted_iota(jnp.int32, (N, N), 1)
    eye = (row_i == col_j)
    offdiag = jnp.where(eye, 0.0, 1.0)
    pair_iota = lax.broadcasted_iota(jnp.int32, (N * N, 1), 0)
    pair_offdiag = jnp.where(pair_iota % (N + 1) == 0, 0.0, 1.0)

    x = x0
    for bl in range(N_BLOCKS):
        w1hi = w_ref[bl, 0]
        w1hj = w_ref[bl, 1]
        we2 = w_ref[bl, 2]
        we3 = w_ref[bl, 3]
        vx1 = w_ref[bl, 4]
        vx2 = w_ref[bl, 5]
        uh1h = w_ref[bl, 6]
        uh1m = w_ref[bl, 7]
        uh2 = w_ref[bl, 8]
        uh3 = w_ref[bl, 9]
        be1 = b_ref[bl, 0]
        be2 = b_ref[bl, 1]
        be3 = b_ref[bl, 2]
        cx1 = b_ref[bl, 3]
        cx2 = b_ref[bl, 4]
        dh1 = b_ref[bl, 5]
        dh2 = b_ref[bl, 6]
        dh3 = b_ref[bl, 7]

        gram = _dotx(x, x.T)
        r_col = jnp.sum(x * x, axis=1, keepdims=True)
        r_row = jnp.sum(jnp.where(eye, gram, 0.0), axis=0, keepdims=True)
        sq = jnp.maximum(r_col + r_row - 2.0 * gram, 0.0)

        a_i = _dot(h, w1hi) + _dotx(te, wte_ref[bl]) + be1[None, :]
        b_j = _dot(h, w1hj)
        z3 = (a_i[:, None, :] + b_j[None, :, :]
              + sq[:, :, None] * wsq_ref[bl][None, :, :])
        z = _silu(z3.reshape(N * N, HID))
        z = _silu(_dot(z, we2) + be2[None, :])
        m = _silu(_dot(z, we3) + be3[None, :])

        s = _silu(_dot(m, vx1) + cx1[None, :])
        s = _silu(_dot(s, vx2) + cx2[None, :])
        s = _dot(s, vx3_ref[bl]) + cx3_ref[bl, 0, 0]

        s_mat = s.reshape(N, N)
        norm = jnp.sqrt(sq + 1e-8)
        wmat = s_mat * offdiag / ((norm + 1.0) * (N - 1.0))
        rowsum = jnp.sum(wmat, axis=1, keepdims=True)
        x = x + x * rowsum - _dotx(wmat, x)

        m_agg = jnp.sum((m * pair_offdiag).reshape(N, N, HID), axis=1)

        g = _silu(_dot(h, uh1h) + _dot(m_agg, uh1m) + dh1[None, :])
        g = _silu(_dot(g, uh2) + dh2[None, :])
        h = h + _dot(g, uh3) + dh3[None, :]

    out_ref[0] = x - x0


@functools.partial(jax.jit, static_argnames=())
def kernel(positions, time, node_features, params):
    bsz = positions.shape[0]
    xr = positions.reshape(bsz, N, DIM)
    t3 = time.reshape(bsz, 1, 1)
    nf3 = node_features.reshape(bsz, N, 1).astype(jnp.int32)

    wmats, biases, wte, wsq, vx3, cx3 = [], [], [], [], [], []
    for bl in range(N_BLOCKS):
        (we1, be1), (we2, be2), (we3, be3) = params['phi_e_%d' % bl]
        (vx1, cx1), (vx2, cx2), (v3, c3) = params['phi_x_%d' % bl]
        (uh1, dh1), (uh2, dh2), (uh3, dh3) = params['phi_h_%d' % bl]
        wmats.append(jnp.stack([
            we1[0:HID], we1[HID:2 * HID], we2, we3,
            vx1, vx2, uh1[0:HID], uh1[HID:2 * HID], uh2, uh3]))
        biases.append(jnp.stack([be1, be2, be3, cx1, cx2, dh1, dh2, dh3]))
        wte.append(we1[2 * HID + 1:])
        wsq.append(we1[2 * HID:2 * HID + 1])
        vx3.append(v3)
        cx3.append(c3.reshape(1, 1))
    wstack = jnp.stack(wmats)
    bstack = jnp.stack(biases)
    wte = jnp.stack(wte)
    wsq = jnp.stack(wsq)
    vx3 = jnp.stack(vx3)
    cx3 = jnp.stack(cx3)

    rep2 = lambda shape: pl.BlockSpec(shape, lambda i: (0,) * len(shape))
    out = pl.pallas_call(
        _egnn_kernel,
        grid=(bsz,),
        in_specs=[
            pl.BlockSpec((1, N, DIM), lambda i: (i, 0, 0)),
            pl.BlockSpec((1, 1, 1), lambda i: (i, 0, 0)),
            pl.BlockSpec((1, N, 1), lambda i: (i, 0, 0)),
            rep2((N_FEATURES, HID)),
            rep2((N_BLOCKS, 10, HID, HID)),
            rep2((N_BLOCKS, 8, HID)),
            rep2((N_BLOCKS, TEMB, HID)),
            rep2((N_BLOCKS, 1, HID)),
            rep2((N_BLOCKS, HID, 1)),
            rep2((N_BLOCKS, 1, 1)),
        ],
        out_specs=pl.BlockSpec((1, N, DIM), lambda i: (i, 0, 0)),
        out_shape=jax.ShapeDtypeStruct((bsz, N, DIM), jnp.float32),
        compiler_params=pltpu.CompilerParams(
            dimension_semantics=("arbitrary",)),
    )(xr, t3, nf3, params['embed'], wstack, bstack, wte, wsq, vx3, cx3)
    return out.reshape(bsz, N * DIM)


# speedup vs baseline: 2.4981x; 1.3951x over previous
import functools

import jax
import jax.numpy as jnp
from jax import lax
from jax.experimental import pallas as pl
from jax.experimental.pallas import tpu as pltpu

B = 64
N = 64
DIM = 3
N_FEATURES = 16
HID = 64
TEMB = 32
N_BLOCKS = 2
G = 2

_LOG1E4 = 9.210340371976184


def _silu(x):
    return (0.5 * x) * (1.0 + jnp.tanh(0.5 * x))


def _dot(a, b):
    return jnp.dot(a, b, preferred_element_type=jnp.float32)


def _dotx(a, b):
    return jnp.dot(a, b, preferred_element_type=jnp.float32,
                   precision=jax.lax.Precision.HIGHEST)


def _egnn_kernel(x_ref, t_ref, nf_ref, emb_ref, w_ref, b_ref, wte_ref,
                 wsq_ref, vx3_ref, cx3_ref, amat_ref, bmask_ref, out_ref):
    x0 = x_ref[...].reshape(G * N, DIM)
    nf = nf_ref[...].reshape(G * N, 1)

    feat_iota = lax.broadcasted_iota(jnp.int32, (G * N, N_FEATURES), 1)
    onehot = (nf == feat_iota).astype(jnp.float32)
    h = _dotx(onehot, emb_ref[:, :])

    tcol = t_ref[...].reshape(G, 1).astype(jnp.float32)
    lane = lax.broadcasted_iota(jnp.int32, (G, TEMB), 1)
    half = TEMB // 2
    k = jnp.where(lane < half, lane, lane - half).astype(jnp.float32)
    freq = jnp.exp(k * (-_LOG1E4 / (half - 1)))
    arg = (tcol * 1000.0) * freq
    te = jnp.where(lane < half, jnp.sin(arg), jnp.cos(arg))

    row_i = lax.broadcasted_iota(jnp.int32, (N, N), 0)
    col_j = lax.broadcasted_iota(jnp.int32, (N, N), 1)
    eye = (row_i == col_j)
    offdiag = jnp.where(eye, 0.0, 1.0)
    pair_iota = lax.broadcasted_iota(jnp.int32, (N * N, 1), 0)
    pair_offdiag = jnp.where(pair_iota % (N + 1) == 0, 0.0, 1.0)

    x = x0
    for bl in range(N_BLOCKS):
        w1hi = w_ref[bl, 0]
        w1hj = w_ref[bl, 1]
        we2 = w_ref[bl, 2]
        we3 = w_ref[bl, 3]
        vx1 = w_ref[bl, 4]
        vx2 = w_ref[bl, 5]
        uh1h = w_ref[bl, 6]
        uh1m = w_ref[bl, 7]
        uh2 = w_ref[bl, 8]
        uh3 = w_ref[bl, 9]
        be1 = b_ref[bl, 0]
        be2 = b_ref[bl, 1]
        be3 = b_ref[bl, 2]
        cx1 = b_ref[bl, 3]
        cx2 = b_ref[bl, 4]
        dh1 = b_ref[bl, 5]
        dh2 = b_ref[bl, 6]
        dh3 = b_ref[bl, 7]

        gram_full = _dotx(x, x.T)
        r_col = jnp.sum(x * x, axis=1, keepdims=True)

        a_i = _dot(h, w1hi) + be1[None, :]
        b_j = _dot(h, w1hj)
        tev = _dotx(te, wte_ref[bl])
        wsqv = wsq_ref[bl]

        z_parts, sqs = [], []
        for g in range(G):
            sl = slice(g * N, (g + 1) * N)
            gram = gram_full[sl, sl]
            rc = r_col[sl]
            rr = jnp.sum(jnp.where(eye, gram, 0.0), axis=0, keepdims=True)
            sq = jnp.maximum(rc + rr - 2.0 * gram, 0.0)
            sqs.append(sq)
            a_g = a_i[sl] + tev[g:g + 1]
            z3 = (a_g[:, None, :] + b_j[sl][None, :, :]
                  + sq[:, :, None] * wsqv[None, :, :])
            z_parts.append(z3.reshape(N * N, HID))
        z = _silu(jnp.concatenate(z_parts, axis=0))
        z = _silu(_dot(z, we2) + be2[None, :])
        m = _silu(_dot(z, we3) + be3[None, :])

        s = _silu(_dot(m, vx1) + cx1[None, :])
        s = _silu(_dot(s, vx2) + cx2[None, :])
        s = _dot(s, vx3_ref[bl]) + cx3_ref[bl, 0, 0]

        xnew_parts, magg_parts = [], []
        for g in range(G):
            sl = slice(g * N, (g + 1) * N)
            pl_ = slice(g * N * N, (g + 1) * N * N)
            s_mat = _dot(amat_ref[:, :], s[pl_] * bmask_ref[:, :])
            norm = jnp.sqrt(sqs[g] + 1e-8)
            wmat = s_mat * offdiag / ((norm + 1.0) * (N - 1.0))
            rowsum = jnp.sum(wmat, axis=1, keepdims=True)
            xg = x[sl]
            xnew_parts.append(xg + xg * rowsum - _dotx(wmat, xg))
            magg_parts.append(_dot(amat_ref[:, :], m[pl_] * pair_offdiag))
        x = jnp.concatenate(xnew_parts, axis=0)
        m_agg = jnp.concatenate(magg_parts, axis=0)

        g_ = _silu(_dot(h, uh1h) + _dot(m_agg, uh1m) + dh1[None, :])
        g_ = _silu(_dot(g_, uh2) + dh2[None, :])
        h = h + _dot(g_, uh3) + dh3[None, :]

    out_ref[...] = (x - x0).reshape(G, N, DIM)


@functools.partial(jax.jit, static_argnames=())
def kernel(positions, time, node_features, params):
    bsz = positions.shape[0]
    xr = positions.reshape(bsz, N, DIM)
    t3 = time.reshape(bsz, 1, 1)
    nf3 = node_features.reshape(bsz, N, 1).astype(jnp.int32)

    wmats, biases, wte, wsq, vx3, cx3 = [], [], [], [], [], []
    for bl in range(N_BLOCKS):
        (we1, be1), (we2, be2), (we3, be3) = params['phi_e_%d' % bl]
        (vx1, cx1), (vx2, cx2), (v3, c3) = params['phi_x_%d' % bl]
        (uh1, dh1), (uh2, dh2), (uh3, dh3) = params['phi_h_%d' % bl]
        wmats.append(jnp.stack([
            we1[0:HID], we1[HID:2 * HID], we2, we3,
            vx1, vx2, uh1[0:HID], uh1[HID:2 * HID], uh2, uh3]))
        biases.append(jnp.stack([be1, be2, be3, cx1, cx2, dh1, dh2, dh3]))
        wte.append(we1[2 * HID + 1:])
        wsq.append(we1[2 * HID:2 * HID + 1])
        vx3.append(v3)
        cx3.append(c3.reshape(1, 1))
    wstack = jnp.stack(wmats)
    bstack = jnp.stack(biases)
    wte = jnp.stack(wte)
    wsq = jnp.stack(wsq)
    vx3 = jnp.stack(vx3)
    cx3 = jnp.stack(cx3)

    pidx = jnp.arange(N * N, dtype=jnp.int32)
    amat = (pidx[None, :] // N == jnp.arange(N, dtype=jnp.int32)[:, None]
            ).astype(jnp.float32)
    bmask = (pidx[:, None] % N == jnp.arange(N, dtype=jnp.int32)[None, :]
             ).astype(jnp.float32)

    rep2 = lambda shape: pl.BlockSpec(shape, lambda i: (0,) * len(shape))
    out = pl.pallas_call(
        _egnn_kernel,
        grid=(bsz // G,),
        in_specs=[
            pl.BlockSpec((G, N, DIM), lambda i: (i, 0, 0)),
            pl.BlockSpec((G, 1, 1), lambda i: (i, 0, 0)),
            pl.BlockSpec((G, N, 1), lambda i: (i, 0, 0)),
            rep2((N_FEATURES, HID)),
            rep2((N_BLOCKS, 10, HID, HID)),
            rep2((N_BLOCKS, 8, HID)),
            rep2((N_BLOCKS, TEMB, HID)),
            rep2((N_BLOCKS, 1, HID)),
            rep2((N_BLOCKS, HID, 1)),
            rep2((N_BLOCKS, 1, 1)),
            rep2((N, N * N)),
            rep2((N * N, N)),
        ],
        out_specs=pl.BlockSpec((G, N, DIM), lambda i: (i, 0, 0)),
        out_shape=jax.ShapeDtypeStruct((bsz, N, DIM), jnp.float32),
        compiler_params=pltpu.CompilerParams(
            dimension_semantics=("arbitrary",)),
    )(xr, t3, nf3, params['embed'], wstack, bstack, wte, wsq, vx3, cx3,
      amat, bmask)
    return out.reshape(bsz, N * DIM)


# speedup vs baseline: 2.7213x; 1.0894x over previous
import functools

import jax
import jax.numpy as jnp
from jax import lax
from jax.experimental import pallas as pl
from jax.experimental.pallas import tpu as pltpu

B = 64
N = 64
DIM = 3
N_FEATURES = 16
HID = 64
TEMB = 32
N_BLOCKS = 2
G = 4

_LOG1E4 = 9.210340371976184


def _silu(x):
    return (0.5 * x) * (1.0 + jnp.tanh(0.5 * x))


def _dot(a, b):
    return jnp.dot(a, b, preferred_element_type=jnp.float32)


def _dotx(a, b):
    return jnp.dot(a, b, preferred_element_type=jnp.float32,
                   precision=jax.lax.Precision.HIGHEST)


def _egnn_kernel(x_ref, t_ref, nf_ref, emb_ref, w_ref, b_ref, wte_ref,
                 wsq_ref, vx3_ref, cx3_ref, amat_ref, bmask_ref, out_ref):
    x0 = x_ref[...].reshape(G * N, DIM)
    nf = nf_ref[...].reshape(G * N, 1)

    feat_iota = lax.broadcasted_iota(jnp.int32, (G * N, N_FEATURES), 1)
    onehot = (nf == feat_iota).astype(jnp.float32)
    h = _dotx(onehot, emb_ref[:, :])

    tcol = t_ref[...].reshape(G, 1).astype(jnp.float32)
    lane = lax.broadcasted_iota(jnp.int32, (G, TEMB), 1)
    half = TEMB // 2
    k = jnp.where(lane < half, lane, lane - half).astype(jnp.float32)
    freq = jnp.exp(k * (-_LOG1E4 / (half - 1)))
    arg = (tcol * 1000.0) * freq
    te = jnp.where(lane < half, jnp.sin(arg), jnp.cos(arg))

    row_i = lax.broadcasted_iota(jnp.int32, (N, N), 0)
    col_j = lax.broadcasted_iota(jnp.int32, (N, N), 1)
    eye = (row_i == col_j)
    offdiag = jnp.where(eye, 0.0, 1.0)
    pair_iota = lax.broadcasted_iota(jnp.int32, (N * N, 1), 0)
    pair_offdiag = jnp.where(pair_iota % (N + 1) == 0, 0.0, 1.0)

    x = x0
    for bl in range(N_BLOCKS):
        w1hi = w_ref[bl, 0]
        w1hj = w_ref[bl, 1]
        we2 = w_ref[bl, 2]
        we3 = w_ref[bl, 3]
        vx1 = w_ref[bl, 4]
        vx2 = w_ref[bl, 5]
        uh1h = w_ref[bl, 6]
        uh1m = w_ref[bl, 7]
        uh2 = w_ref[bl, 8]
        uh3 = w_ref[bl, 9]
        be1 = b_ref[bl, 0]
        be2 = b_ref[bl, 1]
        be3 = b_ref[bl, 2]
        cx1 = b_ref[bl, 3]
        cx2 = b_ref[bl, 4]
        dh1 = b_ref[bl, 5]
        dh2 = b_ref[bl, 6]
        dh3 = b_ref[bl, 7]

        gram_full = _dotx(x, x.T)
        r_col = jnp.sum(x * x, axis=1, keepdims=True)

        a_i = _dot(h, w1hi) + be1[None, :]
        b_j = _dot(h, w1hj)
        tev = _dotx(te, wte_ref[bl])
        wsqv = wsq_ref[bl]

        z_parts, sqs = [], []
        for g in range(G):
            sl = slice(g * N, (g + 1) * N)
            gram = gram_full[sl, sl]
            rc = r_col[sl]
            rr = jnp.sum(jnp.where(eye, gram, 0.0), axis=0, keepdims=True)
            sq = jnp.maximum(rc + rr - 2.0 * gram, 0.0)
            sqs.append(sq)
            a_g = a_i[sl] + tev[g:g + 1]
            z3 = (a_g[:, None, :] + b_j[sl][None, :, :]
                  + sq[:, :, None] * wsqv[None, :, :])
            z_parts.append(z3.reshape(N * N, HID))
        z = _silu(jnp.concatenate(z_parts, axis=0))
        z = _silu(_dot(z, we2) + be2[None, :])
        m = _silu(_dot(z, we3) + be3[None, :])

        s = _silu(_dot(m, vx1) + cx1[None, :])
        s = _silu(_dot(s, vx2) + cx2[None, :])
        s = _dot(s, vx3_ref[bl]) + cx3_ref[bl, 0, 0]

        xnew_parts, magg_parts = [], []
        for g in range(G):
            sl = slice(g * N, (g + 1) * N)
            pl_ = slice(g * N * N, (g + 1) * N * N)
            s_mat = _dot(amat_ref[:, :], s[pl_] * bmask_ref[:, :])
            norm = jnp.sqrt(sqs[g] + 1e-8)
            wmat = s_mat * offdiag / ((norm + 1.0) * (N - 1.0))
            rowsum = jnp.sum(wmat, axis=1, keepdims=True)
            xg = x[sl]
            xnew_parts.append(xg + xg * rowsum - _dotx(wmat, xg))
            magg_parts.append(_dot(amat_ref[:, :], m[pl_] * pair_offdiag))
        x = jnp.concatenate(xnew_parts, axis=0)
        m_agg = jnp.concatenate(magg_parts, axis=0)

        g_ = _silu(_dot(h, uh1h) + _dot(m_agg, uh1m) + dh1[None, :])
        g_ = _silu(_dot(g_, uh2) + dh2[None, :])
        h = h + _dot(g_, uh3) + dh3[None, :]

    out_ref[...] = (x - x0).reshape(G, N, DIM)


@functools.partial(jax.jit, static_argnames=())
def kernel(positions, time, node_features, params):
    bsz = positions.shape[0]
    xr = positions.reshape(bsz, N, DIM)
    t3 = time.reshape(bsz, 1, 1)
    nf3 = node_features.reshape(bsz, N, 1).astype(jnp.int32)

    wmats, biases, wte, wsq, vx3, cx3 = [], [], [], [], [], []
    for bl in range(N_BLOCKS):
        (we1, be1), (we2, be2), (we3, be3) = params['phi_e_%d' % bl]
        (vx1, cx1), (vx2, cx2), (v3, c3) = params['phi_x_%d' % bl]
        (uh1, dh1), (uh2, dh2), (uh3, dh3) = params['phi_h_%d' % bl]
        wmats.append(jnp.stack([
            we1[0:HID], we1[HID:2 * HID], we2, we3,
            vx1, vx2, uh1[0:HID], uh1[HID:2 * HID], uh2, uh3]))
        biases.append(jnp.stack([be1, be2, be3, cx1, cx2, dh1, dh2, dh3]))
        wte.append(we1[2 * HID + 1:])
        wsq.append(we1[2 * HID:2 * HID + 1])
        vx3.append(v3)
        cx3.append(c3.reshape(1, 1))
    wstack = jnp.stack(wmats)
    bstack = jnp.stack(biases)
    wte = jnp.stack(wte)
    wsq = jnp.stack(wsq)
    vx3 = jnp.stack(vx3)
    cx3 = jnp.stack(cx3)

    pidx = jnp.arange(N * N, dtype=jnp.int32)
    amat = (pidx[None, :] // N == jnp.arange(N, dtype=jnp.int32)[:, None]
            ).astype(jnp.float32)
    bmask = (pidx[:, None] % N == jnp.arange(N, dtype=jnp.int32)[None, :]
             ).astype(jnp.float32)

    rep2 = lambda shape: pl.BlockSpec(shape, lambda i: (0,) * len(shape))
    out = pl.pallas_call(
        _egnn_kernel,
        grid=(bsz // G,),
        in_specs=[
            pl.BlockSpec((G, N, DIM), lambda i: (i, 0, 0)),
            pl.BlockSpec((G, 1, 1), lambda i: (i, 0, 0)),
            pl.BlockSpec((G, N, 1), lambda i: (i, 0, 0)),
            rep2((N_FEATURES, HID)),
            rep2((N_BLOCKS, 10, HID, HID)),
            rep2((N_BLOCKS, 8, HID)),
            rep2((N_BLOCKS, TEMB, HID)),
            rep2((N_BLOCKS, 1, HID)),
            rep2((N_BLOCKS, HID, 1)),
            rep2((N_BLOCKS, 1, 1)),
            rep2((N, N * N)),
            rep2((N * N, N)),
        ],
        out_specs=pl.BlockSpec((G, N, DIM), lambda i: (i, 0, 0)),
        out_shape=jax.ShapeDtypeStruct((bsz, N, DIM), jnp.float32),
        compiler_params=pltpu.CompilerParams(
            dimension_semantics=("arbitrary",)),
    )(xr, t3, nf3, params['embed'], wstack, bstack, wte, wsq, vx3, cx3,
      amat, bmask)
    return out.reshape(bsz, N * DIM)
